# baseline (speedup 1.0000x reference)
import jax
import jax.numpy as jnp
from jax import lax
from jax.experimental import pallas as pl
from jax.experimental.pallas import tpu as pltpu
from jax.experimental.pallas import tpu_sc as plsc

N = 10000
E = 320000
D = 128

C = 125
NSUB = 16
NCORE = 2
NW = NSUB * NCORE
CPW = 80
PH = 2
PC = CPW // PH
CAP = NW * CPW * C
NPAD = 10112
RPS = NPAD // NSUB
DEGW = 128

_mesh = plsc.VectorSubcoreMesh(core_axis_name="c", subcore_axis_name="s")


def _agg_body(x_hbm, src_hbm, dst_hbm, z_hbm, out_hbm,
              src_v, dst_v, rows_a, rows_b, acc_s, sem_a, sem_b):
    cid = lax.axis_index("c")
    sid = lax.axis_index("s")
    wid = sid * NCORE + cid

    pltpu.sync_copy(z_hbm, acc_s.at[pl.ds(sid * RPS, RPS)])
    plsc.subcore_barrier()

    bufs = (rows_a, rows_b)
    sems = (sem_a, sem_b)

    for p in range(PH):
        pltpu.sync_copy(src_hbm.at[wid, pl.ds(p * PC, PC)], src_v)
        pltpu.sync_copy(dst_hbm.at[wid, pl.ds(p * PC, PC)], dst_v)

        pltpu.async_copy(x_hbm.at[src_v.at[0]], rows_a, sem_a)
        pltpu.async_copy(x_hbm.at[src_v.at[1]], rows_b, sem_b)

        def body(i, carry):
            j = i * 2
            for b in range(2):
                buf, sem = bufs[b], sems[b]
                pltpu.make_async_copy(
                    x_hbm.at[src_v.at[j + b]], buf, sem).wait()
                pltpu.sync_copy(buf, acc_s.at[dst_v.at[j + b]], add=True)
                pltpu.async_copy(x_hbm.at[src_v.at[j + 2 + b]], buf, sem)
            return carry

        lax.fori_loop(0, (PC - 2) // 2, body, 0)
        for b in range(2):
            pltpu.make_async_copy(
                x_hbm.at[src_v.at[PC - 2 + b]], bufs[b], sems[b]).wait()
            pltpu.sync_copy(bufs[b], acc_s.at[dst_v.at[PC - 2 + b]], add=True)

    plsc.subcore_barrier()

    pltpu.sync_copy(acc_s.at[pl.ds(sid * RPS, RPS)],
                    out_hbm.at[cid, pl.ds(sid * RPS, RPS)])


_sc_agg = pl.kernel(
    _agg_body,
    out_type=jax.ShapeDtypeStruct((NCORE, NPAD, D), jnp.float32),
    mesh=_mesh,
    scratch_types=[
        pltpu.VMEM((PC, C), jnp.int32),
        pltpu.VMEM((PC, C), jnp.int32),
        pltpu.VMEM((C, D), jnp.float32),
        pltpu.VMEM((C, D), jnp.float32),
        pltpu.VMEM_SHARED((NPAD, D), jnp.float32),
        pltpu.SemaphoreType.DMA,
        pltpu.SemaphoreType.DMA,
    ],
)


def _deg_body(dst_hbm, zd_hbm, ones_hbm, deg_hbm, dst_v, ones_v, deg_s):
    cid = lax.axis_index("c")
    sid = lax.axis_index("s")
    wid = sid * NCORE + cid

    pltpu.sync_copy(zd_hbm, deg_s.at[pl.ds(sid * RPS, RPS)])
    pltpu.sync_copy(dst_hbm.at[wid], dst_v)
    pltpu.sync_copy(ones_hbm, ones_v)
    plsc.subcore_barrier()

    def body(j, carry):
        pltpu.sync_copy(ones_v, deg_s.at[dst_v.at[j]], add=True)
        return carry

    lax.fori_loop(0, CPW, body, 0)
    plsc.subcore_barrier()

    pltpu.sync_copy(deg_s.at[pl.ds(sid * RPS, RPS)],
                    deg_hbm.at[cid, pl.ds(sid * RPS, RPS)])


_sc_deg = pl.kernel(
    _deg_body,
    out_type=jax.ShapeDtypeStruct((NCORE, NPAD, DEGW), jnp.float32),
    mesh=_mesh,
    scratch_types=[
        pltpu.VMEM((CPW, C), jnp.int32),
        pltpu.VMEM((C, DEGW), jnp.float32),
        pltpu.VMEM_SHARED((NPAD, DEGW), jnp.float32),
    ],
)

BLK = 1000


def _tc0_body(pn_ref, pd_ref, x_ref, wt_ref, b_ref, out_h, out_r):
    s = pn_ref[0] + pn_ref[1] + x_ref[...]
    deg = pd_ref[0, :, 0] + pd_ref[1, :, 0] + 1.0
    hn = s / deg[:, None]
    h = jnp.dot(hn, wt_ref[...], preferred_element_type=jnp.float32) + b_ref[...]
    out_h[...] = h
    out_r[...] = jnp.maximum(h, 0.0)


def _tc1_body(pn_ref, pd_ref, x_ref, wt_ref, b_ref, out_ref):
    s = pn_ref[0] + pn_ref[1] + x_ref[...]
    deg = pd_ref[0, :, 0] + pd_ref[1, :, 0] + 1.0
    hn = s / deg[:, None]
    out_ref[...] = jnp.dot(hn, wt_ref[...],
                           preferred_element_type=jnp.float32) + b_ref[...]


_x_spec = pl.BlockSpec((BLK, D), lambda i: (i, 0))
_w_spec = pl.BlockSpec((D, D), lambda i: (0, 0))
_b_spec = pl.BlockSpec((1, D), lambda i: (0, 0))
_p_spec = pl.BlockSpec((NCORE, BLK, D), lambda i: (0, i, 0))
_d_spec = pl.BlockSpec((NCORE, BLK, DEGW), lambda i: (0, i, 0))

_tc_layer0 = pl.pallas_call(
    _tc0_body,
    grid=(N // BLK,),
    in_specs=[_p_spec, _d_spec, _x_spec, _w_spec, _b_spec],
    out_specs=[_x_spec, _x_spec],
    out_shape=[jax.ShapeDtypeStruct((N, D), jnp.float32)] * 2,
)

_tc_layer1 = pl.pallas_call(
    _tc1_body,
    grid=(N // BLK,),
    in_specs=[_p_spec, _d_spec, _x_spec, _w_spec, _b_spec],
    out_specs=[_x_spec],
    out_shape=[jax.ShapeDtypeStruct((N, D), jnp.float32)],
)


@jax.jit
def kernel(blocks, feats, W1, b1, W2, b2):
    srcp = blocks[0].reshape(NW, CPW, C)
    dstp = blocks[1].reshape(NW, CPW, C)
    z = jnp.zeros((RPS, D), jnp.float32)
    zdeg = jnp.zeros((RPS, DEGW), jnp.float32)
    ones = jnp.ones((C, DEGW), jnp.float32)

    degp = _sc_deg(dstp, zdeg, ones)
    p0 = _sc_agg(feats, srcp, dstp, z)
    h_hidden, h_relu = _tc_layer0(p0, degp, feats, W1.T, b1.reshape(1, D))
    p1 = _sc_agg(h_relu, srcp, dstp, z)
    h2, = _tc_layer1(p1, degp, h_relu, W2.T, b2.reshape(1, D))
    return (h_hidden, h2)


# speedup vs baseline: 1.0052x; 1.0052x over previous
import jax
import jax.numpy as jnp
from jax import lax
from jax.experimental import pallas as pl
from jax.experimental.pallas import tpu as pltpu
from jax.experimental.pallas import tpu_sc as plsc

N = 10000
E = 320000
D = 128

C = 125
NSUB = 16
NCORE = 2
NW = NSUB * NCORE
CPW = 80
PH = 2
PC = CPW // PH
CAP = NW * CPW * C
NPAD = 10112
RPS = NPAD // NSUB
DEGW = 128

_mesh = plsc.VectorSubcoreMesh(core_axis_name="c", subcore_axis_name="s")


def _agg_body(x_hbm, src_hbm, dst_hbm, z_hbm, out_hbm,
              src_v, dst_v, rows_a, rows_b, acc_s, sem_a, sem_b):
    cid = lax.axis_index("c")
    sid = lax.axis_index("s")
    wid = sid * NCORE + cid

    pltpu.sync_copy(z_hbm, acc_s.at[pl.ds(sid * RPS, RPS)])
    plsc.subcore_barrier()

    bufs = (rows_a, rows_b)
    sems = (sem_a, sem_b)

    for p in range(PH):
        pltpu.sync_copy(src_hbm.at[wid, pl.ds(p * PC, PC)], src_v)
        pltpu.sync_copy(dst_hbm.at[wid, pl.ds(p * PC, PC)], dst_v)

        pltpu.async_copy(x_hbm.at[src_v.at[0]], rows_a, sem_a)
        pltpu.async_copy(x_hbm.at[src_v.at[1]], rows_b, sem_b)

        def body(i, carry):
            j = i * 2
            for b in range(2):
                buf, sem = bufs[b], sems[b]
                pltpu.make_async_copy(
                    x_hbm.at[src_v.at[j + b]], buf, sem).wait()
                pltpu.sync_copy(buf, acc_s.at[dst_v.at[j + b]], add=True)
                pltpu.async_copy(x_hbm.at[src_v.at[j + 2 + b]], buf, sem)
            return carry

        lax.fori_loop(0, (PC - 2) // 2, body, 0)
        for b in range(2):
            pltpu.make_async_copy(
                x_hbm.at[src_v.at[PC - 2 + b]], bufs[b], sems[b]).wait()
            pltpu.sync_copy(bufs[b], acc_s.at[dst_v.at[PC - 2 + b]], add=True)

    plsc.subcore_barrier()

    pltpu.sync_copy(acc_s.at[pl.ds(sid * RPS, RPS)],
                    out_hbm.at[cid, pl.ds(sid * RPS, RPS)])


_sc_agg = pl.kernel(
    _agg_body,
    out_type=jax.ShapeDtypeStruct((NCORE, NPAD, D), jnp.float32),
    mesh=_mesh,
    scratch_types=[
        pltpu.VMEM((PC, C), jnp.int32),
        pltpu.VMEM((PC, C), jnp.int32),
        pltpu.VMEM((C, D), jnp.float32),
        pltpu.VMEM((C, D), jnp.float32),
        pltpu.VMEM_SHARED((NPAD, D), jnp.float32),
        pltpu.SemaphoreType.DMA,
        pltpu.SemaphoreType.DMA,
    ],
)


def _deg_body(dst_hbm, zd_hbm, ones_hbm, deg_hbm, dst_v, ones_v, deg_s):
    cid = lax.axis_index("c")
    sid = lax.axis_index("s")
    wid = sid * NCORE + cid

    pltpu.sync_copy(zd_hbm, deg_s.at[pl.ds(sid * RPS, RPS)])
    pltpu.sync_copy(dst_hbm.at[wid], dst_v)
    pltpu.sync_copy(ones_hbm, ones_v)
    plsc.subcore_barrier()

    def body(j, carry):
        pltpu.sync_copy(ones_v, deg_s.at[dst_v.at[j]], add=True)
        return carry

    lax.fori_loop(0, CPW, body, 0)
    plsc.subcore_barrier()

    pltpu.sync_copy(deg_s.at[pl.ds(sid * RPS, RPS)],
                    deg_hbm.at[cid, pl.ds(sid * RPS, RPS)])


_sc_deg = pl.kernel(
    _deg_body,
    out_type=jax.ShapeDtypeStruct((NCORE, NPAD, DEGW), jnp.float32),
    mesh=_mesh,
    scratch_types=[
        pltpu.VMEM((CPW, C), jnp.int32),
        pltpu.VMEM((C, DEGW), jnp.float32),
        pltpu.VMEM_SHARED((NPAD, DEGW), jnp.float32),
    ],
)

BLK = 1000


def _tc0_body(pn_ref, pd_ref, x_ref, wt_ref, b_ref, out_h, out_r):
    s = pn_ref[0] + pn_ref[1] + x_ref[...]
    deg = pd_ref[0, :, 0] + pd_ref[1, :, 0] + 1.0
    hn = s / deg[:, None]
    h = jnp.dot(hn, wt_ref[...], preferred_element_type=jnp.float32) + b_ref[...]
    out_h[...] = h
    out_r[...] = jnp.maximum(h, 0.0)


def _tc1_body(pn_ref, pd_ref, x_ref, wt_ref, b_ref, out_ref):
    s = pn_ref[0] + pn_ref[1] + x_ref[...]
    deg = pd_ref[0, :, 0] + pd_ref[1, :, 0] + 1.0
    hn = s / deg[:, None]
    out_ref[...] = jnp.dot(hn, wt_ref[...],
                           preferred_element_type=jnp.float32) + b_ref[...]


_x_spec = pl.BlockSpec((BLK, D), lambda i: (i, 0))
_w_spec = pl.BlockSpec((D, D), lambda i: (0, 0))
_b_spec = pl.BlockSpec((1, D), lambda i: (0, 0))
_p_spec = pl.BlockSpec((NCORE, BLK, D), lambda i: (0, i, 0))
_d_spec = pl.BlockSpec((NCORE, BLK, DEGW), lambda i: (0, i, 0))

_tc_layer0 = pl.pallas_call(
    _tc0_body,
    grid=(N // BLK,),
    in_specs=[_p_spec, _d_spec, _x_spec, _w_spec, _b_spec],
    out_specs=[_x_spec, _x_spec],
    out_shape=[jax.ShapeDtypeStruct((N, D), jnp.float32)] * 2,
)

_tc_layer1 = pl.pallas_call(
    _tc1_body,
    grid=(N // BLK,),
    in_specs=[_p_spec, _d_spec, _x_spec, _w_spec, _b_spec],
    out_specs=[_x_spec],
    out_shape=[jax.ShapeDtypeStruct((N, D), jnp.float32)],
)


@jax.jit
def kernel(blocks, feats, W1, b1, W2, b2):
    srcp = blocks[0].reshape(NW, CPW, C)
    dstp = blocks[1].reshape(NW, CPW, C)
    z = jnp.zeros((RPS, D), jnp.float32)
    zdeg = jnp.zeros((RPS, DEGW), jnp.float32)
    ones = jnp.ones((C, DEGW), jnp.float32)

    degp = _sc_deg(dstp, zdeg, ones)
    z0 = z + jnp.minimum(degp[0, 0, 0], 0.0)
    p0 = _sc_agg(feats, srcp, dstp, z0)
    h_hidden, h_relu = _tc_layer0(p0, degp, feats, W1.T, b1.reshape(1, D))
    p1 = _sc_agg(h_relu, srcp, dstp, z)
    h2, = _tc_layer1(p1, degp, h_relu, W2.T, b2.reshape(1, D))
    return (h_hidden, h2)


# speedup vs baseline: 1.0231x; 1.0178x over previous
import jax
import jax.numpy as jnp
from jax import lax
from jax.experimental import pallas as pl
from jax.experimental.pallas import tpu as pltpu
from jax.experimental.pallas import tpu_sc as plsc

N = 10000
E = 320000
D = 128

C = 125
NSUB = 16
NCORE = 2
NW = NSUB * NCORE
CPW = 80
PH = 2
PC = CPW // PH
CAP = NW * CPW * C
NPAD = 10112
RPS = NPAD // NSUB
DEGW = 128

_mesh = plsc.VectorSubcoreMesh(core_axis_name="c", subcore_axis_name="s")


def _agg_body(x_hbm, e_hbm, z_hbm, out_hbm,
              src_v, dst_v, rows_a, rows_b, acc_s, sem_a, sem_b):
    cid = lax.axis_index("c")
    sid = lax.axis_index("s")
    wid = sid * NCORE + cid

    pltpu.sync_copy(z_hbm, acc_s.at[pl.ds(sid * RPS, RPS)])
    plsc.subcore_barrier()

    bufs = (rows_a, rows_b)
    sems = (sem_a, sem_b)

    for p in range(PH):
        pltpu.sync_copy(e_hbm.at[0, wid, pl.ds(p * PC, PC)], src_v)
        pltpu.sync_copy(e_hbm.at[1, wid, pl.ds(p * PC, PC)], dst_v)

        pltpu.async_copy(x_hbm.at[src_v.at[0]], rows_a, sem_a)
        pltpu.async_copy(x_hbm.at[src_v.at[1]], rows_b, sem_b)

        def body(i, carry):
            j = i * 2
            for b in range(2):
                buf, sem = bufs[b], sems[b]
                pltpu.make_async_copy(
                    x_hbm.at[src_v.at[j + b]], buf, sem).wait()
                pltpu.sync_copy(buf, acc_s.at[dst_v.at[j + b]], add=True)
                pltpu.async_copy(x_hbm.at[src_v.at[j + 2 + b]], buf, sem)
            return carry

        lax.fori_loop(0, (PC - 2) // 2, body, 0)
        for b in range(2):
            pltpu.make_async_copy(
                x_hbm.at[src_v.at[PC - 2 + b]], bufs[b], sems[b]).wait()
            pltpu.sync_copy(bufs[b], acc_s.at[dst_v.at[PC - 2 + b]], add=True)

    plsc.subcore_barrier()

    pltpu.sync_copy(acc_s.at[pl.ds(sid * RPS, RPS)],
                    out_hbm.at[cid, pl.ds(sid * RPS, RPS)])


_sc_agg = pl.kernel(
    _agg_body,
    out_type=jax.ShapeDtypeStruct((NCORE, NPAD, D), jnp.float32),
    mesh=_mesh,
    scratch_types=[
        pltpu.VMEM((PC, C), jnp.int32),
        pltpu.VMEM((PC, C), jnp.int32),
        pltpu.VMEM((C, D), jnp.float32),
        pltpu.VMEM((C, D), jnp.float32),
        pltpu.VMEM_SHARED((NPAD, D), jnp.float32),
        pltpu.SemaphoreType.DMA,
        pltpu.SemaphoreType.DMA,
    ],
)


def _deg_body(e_hbm, zd_hbm, ones_hbm, deg_hbm, dst_v, ones_v, deg_s):
    cid = lax.axis_index("c")
    sid = lax.axis_index("s")
    wid = sid * NCORE + cid

    pltpu.sync_copy(zd_hbm, deg_s.at[pl.ds(sid * RPS, RPS)])
    pltpu.sync_copy(e_hbm.at[1, wid], dst_v)
    pltpu.sync_copy(ones_hbm, ones_v)
    plsc.subcore_barrier()

    def body(j, carry):
        pltpu.sync_copy(ones_v, deg_s.at[dst_v.at[j]], add=True)
        return carry

    lax.fori_loop(0, CPW, body, 0)
    plsc.subcore_barrier()

    pltpu.sync_copy(deg_s.at[pl.ds(sid * RPS, RPS)],
                    deg_hbm.at[cid, pl.ds(sid * RPS, RPS)])


_sc_deg = pl.kernel(
    _deg_body,
    out_type=jax.ShapeDtypeStruct((NCORE, NPAD, DEGW), jnp.float32),
    mesh=_mesh,
    scratch_types=[
        pltpu.VMEM((CPW, C), jnp.int32),
        pltpu.VMEM((C, DEGW), jnp.float32),
        pltpu.VMEM_SHARED((NPAD, DEGW), jnp.float32),
    ],
)

BLK = 1000


def _tc0_body(pn_ref, pd_ref, x_ref, wt_ref, b_ref, out_h, out_r):
    s = pn_ref[0] + pn_ref[1] + x_ref[...]
    deg = pd_ref[0, :, 0] + pd_ref[1, :, 0] + 1.0
    hn = s / deg[:, None]
    h = jnp.dot(hn, wt_ref[...], preferred_element_type=jnp.float32) + b_ref[...]
    out_h[...] = h
    out_r[...] = jnp.maximum(h, 0.0)


def _tc1_body(pn_ref, pd_ref, x_ref, wt_ref, b_ref, out_ref):
    s = pn_ref[0] + pn_ref[1] + x_ref[...]
    deg = pd_ref[0, :, 0] + pd_ref[1, :, 0] + 1.0
    hn = s / deg[:, None]
    out_ref[...] = jnp.dot(hn, wt_ref[...],
                           preferred_element_type=jnp.float32) + b_ref[...]


_x_spec = pl.BlockSpec((BLK, D), lambda i: (i, 0))
_w_spec = pl.BlockSpec((D, D), lambda i: (0, 0))
_b_spec = pl.BlockSpec((1, D), lambda i: (0, 0))
_p_spec = pl.BlockSpec((NCORE, BLK, D), lambda i: (0, i, 0))
_d_spec = pl.BlockSpec((NCORE, BLK, DEGW), lambda i: (0, i, 0))

_tc_layer0 = pl.pallas_call(
    _tc0_body,
    grid=(N // BLK,),
    in_specs=[_p_spec, _d_spec, _x_spec, _w_spec, _b_spec],
    out_specs=[_x_spec, _x_spec],
    out_shape=[jax.ShapeDtypeStruct((N, D), jnp.float32)] * 2,
)

_tc_layer1 = pl.pallas_call(
    _tc1_body,
    grid=(N // BLK,),
    in_specs=[_p_spec, _d_spec, _x_spec, _w_spec, _b_spec],
    out_specs=[_x_spec],
    out_shape=[jax.ShapeDtypeStruct((N, D), jnp.float32)],
)


@jax.jit
def kernel(blocks, feats, W1, b1, W2, b2):
    edges = blocks.reshape(2, NW, CPW, C)
    z = jnp.zeros((RPS, D), jnp.float32)
    zdeg = jnp.zeros((RPS, DEGW), jnp.float32)
    ones = jnp.ones((C, DEGW), jnp.float32)

    degp = _sc_deg(edges, zdeg, ones)
    z0 = z + jnp.minimum(degp[0, 0, 0], 0.0)
    p0 = _sc_agg(feats, edges, z0)
    h_hidden, h_relu = _tc_layer0(p0, degp, feats, W1.T, b1.reshape(1, D))
    p1 = _sc_agg(h_relu, edges, z)
    h2, = _tc_layer1(p1, degp, h_relu, W2.T, b2.reshape(1, D))
    return (h_hidden, h2)


# speedup vs baseline: 1.0272x; 1.0041x over previous
import jax
import jax.numpy as jnp
from jax import lax
from jax.experimental import pallas as pl
from jax.experimental.pallas import tpu as pltpu
from jax.experimental.pallas import tpu_sc as plsc

N = 10000
E = 320000
D = 128

C = 80
NSUB = 16
NCORE = 2
NW = NSUB * NCORE
CPW = 125
PH = 5
PC = CPW // PH
NB = 3
CAP = NW * CPW * C
NPAD = 10112
RPS = NPAD // NSUB
DEGW = 128

_mesh = plsc.VectorSubcoreMesh(core_axis_name="c", subcore_axis_name="s")


def _agg_body(x_hbm, e_hbm, z_hbm, out_hbm,
              src_v, dst_v, rows_a, rows_b, rows_c, acc_s,
              sem_a, sem_b, sem_c):
    cid = lax.axis_index("c")
    sid = lax.axis_index("s")
    wid = sid * NCORE + cid

    pltpu.sync_copy(z_hbm, acc_s.at[pl.ds(sid * RPS, RPS)])
    plsc.subcore_barrier()

    bufs = (rows_a, rows_b, rows_c)
    sems = (sem_a, sem_b, sem_c)

    nloop = (PC - NB) // NB
    tail = PC - NB - nloop * NB
    for p in range(PH):
        pltpu.sync_copy(e_hbm.at[0, wid, p], src_v)
        pltpu.sync_copy(e_hbm.at[1, wid, p], dst_v)

        for b in range(NB):
            pltpu.async_copy(x_hbm.at[src_v.at[b]], bufs[b], sems[b])

        def body(i, carry):
            j = i * NB
            for b in range(NB):
                buf, sem = bufs[b], sems[b]
                pltpu.make_async_copy(
                    x_hbm.at[src_v.at[j + b]], buf, sem).wait()
                pltpu.sync_copy(buf, acc_s.at[dst_v.at[j + b]], add=True)
                pltpu.async_copy(x_hbm.at[src_v.at[j + NB + b]], buf, sem)
            return carry

        lax.fori_loop(0, nloop, body, 0)
        for t in range(nloop * NB, PC):
            b = t % NB
            pltpu.make_async_copy(
                x_hbm.at[src_v.at[t]], bufs[b], sems[b]).wait()
            pltpu.sync_copy(bufs[b], acc_s.at[dst_v.at[t]], add=True)
            if t + NB < PC:
                pltpu.async_copy(
                    x_hbm.at[src_v.at[t + NB]], bufs[b], sems[b])

    plsc.subcore_barrier()

    pltpu.sync_copy(acc_s.at[pl.ds(sid * RPS, RPS)],
                    out_hbm.at[cid, pl.ds(sid * RPS, RPS)])


_sc_agg = pl.kernel(
    _agg_body,
    out_type=jax.ShapeDtypeStruct((NCORE, NPAD, D), jnp.float32),
    mesh=_mesh,
    scratch_types=[
        pltpu.VMEM((PC, C), jnp.int32),
        pltpu.VMEM((PC, C), jnp.int32),
        pltpu.VMEM((C, D), jnp.float32),
        pltpu.VMEM((C, D), jnp.float32),
        pltpu.VMEM((C, D), jnp.float32),
        pltpu.VMEM_SHARED((NPAD, D), jnp.float32),
        pltpu.SemaphoreType.DMA,
        pltpu.SemaphoreType.DMA,
        pltpu.SemaphoreType.DMA,
    ],
)


def _deg_body(e_hbm, zd_hbm, ones_hbm, deg_hbm, dst_v, ones_v, deg_s):
    cid = lax.axis_index("c")
    sid = lax.axis_index("s")
    wid = sid * NCORE + cid

    pltpu.sync_copy(zd_hbm, deg_s.at[pl.ds(sid * RPS, RPS)])
    pltpu.sync_copy(e_hbm.at[1, wid], dst_v)
    pltpu.sync_copy(ones_hbm, ones_v)
    plsc.subcore_barrier()

    for p in range(PH):
        def body(j, carry):
            pltpu.sync_copy(ones_v, deg_s.at[dst_v.at[p, j]], add=True)
            return carry

        lax.fori_loop(0, PC, body, 0)
    plsc.subcore_barrier()

    pltpu.sync_copy(deg_s.at[pl.ds(sid * RPS, RPS)],
                    deg_hbm.at[cid, pl.ds(sid * RPS, RPS)])


_sc_deg = pl.kernel(
    _deg_body,
    out_type=jax.ShapeDtypeStruct((NCORE, NPAD, DEGW), jnp.float32),
    mesh=_mesh,
    scratch_types=[
        pltpu.VMEM((PH, PC, C), jnp.int32),
        pltpu.VMEM((C, DEGW), jnp.float32),
        pltpu.VMEM_SHARED((NPAD, DEGW), jnp.float32),
    ],
)

BLK = 1000


def _tc0_body(pn_ref, pd_ref, x_ref, wt_ref, b_ref, out_h, out_r):
    s = pn_ref[0] + pn_ref[1] + x_ref[...]
    deg = pd_ref[0, :, 0] + pd_ref[1, :, 0] + 1.0
    hn = s / deg[:, None]
    h = jnp.dot(hn, wt_ref[...], preferred_element_type=jnp.float32) + b_ref[...]
    out_h[...] = h
    out_r[...] = jnp.maximum(h, 0.0)


def _tc1_body(pn_ref, pd_ref, x_ref, wt_ref, b_ref, out_ref):
    s = pn_ref[0] + pn_ref[1] + x_ref[...]
    deg = pd_ref[0, :, 0] + pd_ref[1, :, 0] + 1.0
    hn = s / deg[:, None]
    out_ref[...] = jnp.dot(hn, wt_ref[...],
                           preferred_element_type=jnp.float32) + b_ref[...]


_x_spec = pl.BlockSpec((BLK, D), lambda i: (i, 0))
_w_spec = pl.BlockSpec((D, D), lambda i: (0, 0))
_b_spec = pl.BlockSpec((1, D), lambda i: (0, 0))
_p_spec = pl.BlockSpec((NCORE, BLK, D), lambda i: (0, i, 0))
_d_spec = pl.BlockSpec((NCORE, BLK, DEGW), lambda i: (0, i, 0))

_tc_layer0 = pl.pallas_call(
    _tc0_body,
    grid=(N // BLK,),
    in_specs=[_p_spec, _d_spec, _x_spec, _w_spec, _b_spec],
    out_specs=[_x_spec, _x_spec],
    out_shape=[jax.ShapeDtypeStruct((N, D), jnp.float32)] * 2,
)

_tc_layer1 = pl.pallas_call(
    _tc1_body,
    grid=(N // BLK,),
    in_specs=[_p_spec, _d_spec, _x_spec, _w_spec, _b_spec],
    out_specs=[_x_spec],
    out_shape=[jax.ShapeDtypeStruct((N, D), jnp.float32)],
)


@jax.jit
def kernel(blocks, feats, W1, b1, W2, b2):
    edges = blocks.reshape(2, NW, PH, PC, C)
    z = jnp.zeros((RPS, D), jnp.float32)
    zdeg = jnp.zeros((RPS, DEGW), jnp.float32)
    ones = jnp.ones((C, DEGW), jnp.float32)

    degp = _sc_deg(edges, zdeg, ones)
    z0 = z + jnp.minimum(degp[0, 0, 0], 0.0)
    p0 = _sc_agg(feats, edges, z0)
    h_hidden, h_relu = _tc_layer0(p0, degp, feats, W1.T, b1.reshape(1, D))
    p1 = _sc_agg(h_relu, edges, z)
    h2, = _tc_layer1(p1, degp, h_relu, W2.T, b2.reshape(1, D))
    return (h_hidden, h2)
